# baseline (speedup 1.0000x reference)
import functools

import jax
import jax.numpy as jnp
from jax import lax
from jax.experimental import pallas as pl
from jax.experimental.pallas import tpu as pltpu
from jax.experimental.pallas import tpu_sc as plsc

N = 10000
E = 320000
D = 128
NC = 2
NS = 16
NW = NC * NS
KB = 128
NCH = 80
SEG = 16
CHUNKS = NW * NCH
NCH0 = 160
NCH1 = 0
E_PAD = CHUNKS * KB
N_PAD = N + 16
STEP = 624
STRIPE = 640
CW = 128

_mesh = plsc.VectorSubcoreMesh(
    core_axis_name="c", subcore_axis_name="s", num_cores=NC, num_subcores=NS
)


@functools.partial(
    pl.kernel,
    out_type=jax.ShapeDtypeStruct((NC * N, CW), jnp.float32),
    mesh=_mesh,
    scratch_types=[
        pltpu.VMEM((NCH, KB), jnp.int32),
        pltpu.VMEM((KB, CW), jnp.float32),
        pltpu.VMEM_SHARED((N_PAD, CW), jnp.float32),
    ],
)
def _sc_cnt(dst_hbm, z16_hbm, ones_hbm, cnt_out, dst_v, ones_v, cnt_s):
    cid = lax.axis_index("c")
    sid = lax.axis_index("s")
    wid = cid * NS + sid
    r0 = sid * STEP
    pltpu.sync_copy(z16_hbm.at[pl.ds(r0, STRIPE)], cnt_s.at[pl.ds(r0, STRIPE)])
    pltpu.sync_copy(ones_hbm, ones_v)
    pltpu.sync_copy(dst_hbm.at[pl.ds(wid * NCH, NCH)], dst_v)
    plsc.subcore_barrier()

    def step(j, carry):
        pltpu.sync_copy(ones_v, cnt_s.at[dst_v.at[j]], add=True)
        return carry

    lax.fori_loop(0, NCH, step, 0)
    plsc.subcore_barrier()
    o0 = cid * N + r0
    pltpu.sync_copy(cnt_s.at[pl.ds(r0, STRIPE)], cnt_out.at[pl.ds(o0, STRIPE)])


@functools.partial(
    pl.kernel,
    out_type=jax.ShapeDtypeStruct((NC * N, D), jnp.float32),
    mesh=_mesh,
    scratch_types=[
        pltpu.VMEM((SEG, KB), jnp.int32),
        pltpu.VMEM((SEG, KB), jnp.int32),
        pltpu.VMEM((KB, D), jnp.float32),
        pltpu.VMEM((KB, D), jnp.float32),
        pltpu.VMEM_SHARED((N_PAD, D), jnp.float32),
        pltpu.SemaphoreType.DMA,
        pltpu.SemaphoreType.DMA,
        pltpu.SemaphoreType.DMA,
    ],
)
def _sc_agg(x_hbm, src_hbm, dst_hbm, z128_hbm, agg_out, src_seg, dst_seg,
            rows0, rows1, agg_s, gs0, gs1, ss):
    cid = lax.axis_index("c")
    sid = lax.axis_index("s")
    r0 = sid * STEP
    pltpu.sync_copy(z128_hbm.at[pl.ds(r0, STRIPE)], agg_s.at[pl.ds(r0, STRIPE)])
    plsc.subcore_barrier()

    rows = (rows0, rows1)
    gsems = (gs0, gs1)

    def run_chunks(cbase, nseg):
        def seg_body(s, carry):
            base = cbase + s * SEG
            pltpu.sync_copy(src_hbm.at[pl.ds(base, SEG)], src_seg)
            pltpu.sync_copy(dst_hbm.at[pl.ds(base, SEG)], dst_seg)

            def step2(i, c2):
                j0 = 2 * i
                dgs = [
                    pltpu.async_copy(x_hbm.at[src_seg.at[j0 + b]], rows[b],
                                     gsems[b])
                    for b in range(2)
                ]
                dss = []
                for b in range(2):
                    dgs[b].wait()
                    dss.append(
                        pltpu.async_copy(rows[b],
                                         agg_s.at[dst_seg.at[j0 + b]],
                                         ss, add=True))
                for d in dss:
                    d.wait()
                return c2

            lax.fori_loop(0, SEG // 2, step2, 0)
            return carry

        lax.fori_loop(0, nseg, seg_body, 0)

    @pl.when(cid == 0)
    def _():
        run_chunks(sid * NCH0, NCH0 // SEG)

    if NCH1:
        @pl.when(cid == 1)
        def _():
            run_chunks(NS * NCH0 + sid * NCH1, NCH1 // SEG)

    plsc.subcore_barrier()
    o0 = cid * N + r0
    pltpu.sync_copy(agg_s.at[pl.ds(r0, STRIPE)], agg_out.at[pl.ds(o0, STRIPE)])


BR = 2000


def _dense_body(aggA, aggB, cntA, cntB, xin, wlT, b, wrT, o):
    c = cntA[:, 0:1] + cntB[:, 0:1]
    mean = (aggA[...] + aggB[...]) / jnp.maximum(c, 1.0)
    acc = jnp.dot(mean, wlT[...], preferred_element_type=jnp.float32,
                  precision=lax.Precision.HIGHEST)
    acc = acc + jnp.dot(xin[...], wrT[...], preferred_element_type=jnp.float32,
                        precision=lax.Precision.HIGHEST)
    o[...] = jnp.maximum(acc + b[...], 0.0)


def _dense(agg, cnt, xin, wlT, b2d, wrT):
    nb = N // BR
    return pl.pallas_call(
        _dense_body,
        grid=(nb,),
        in_specs=[
            pl.BlockSpec((BR, D), lambda i: (i, 0)),
            pl.BlockSpec((BR, D), lambda i: (i + nb, 0)),
            pl.BlockSpec((BR, CW), lambda i: (i, 0)),
            pl.BlockSpec((BR, CW), lambda i: (i + nb, 0)),
            pl.BlockSpec((BR, D), lambda i: (i, 0)),
            pl.BlockSpec((D, D), lambda i: (0, 0)),
            pl.BlockSpec((1, D), lambda i: (0, 0)),
            pl.BlockSpec((D, D), lambda i: (0, 0)),
        ],
        out_specs=pl.BlockSpec((BR, D), lambda i: (i, 0)),
        out_shape=jax.ShapeDtypeStruct((N, D), jnp.float32),
    )(agg, agg, cnt, cnt, xin, wlT, b2d, wrT)


def kernel(x, edge_index, W1l, b1, W1r, W2l, b2, W2r):
    pad = E_PAD - E
    src = jnp.concatenate([edge_index[0], jnp.zeros((pad,), jnp.int32)])
    dst = jnp.concatenate([edge_index[1], jnp.full((pad,), N, jnp.int32)])
    src3 = src.reshape(CHUNKS, KB)
    dst3 = dst.reshape(CHUNKS, KB)
    z128 = jnp.zeros((N, D), jnp.float32)
    ones = jnp.ones((KB, CW), jnp.float32)

    cnt = _sc_cnt(dst3, z128, ones)
    agg1 = _sc_agg(x, src3, dst3, z128)
    h = _dense(agg1, cnt, x, W1l.T, b1.reshape(1, D), W1r.T)
    agg2 = _sc_agg(h, src3, dst3, z128)
    return _dense(agg2, cnt, h, W2l.T, b2.reshape(1, D), W2r.T)


# speedup vs baseline: 2.9867x; 2.9867x over previous
import functools

import jax
import jax.numpy as jnp
from jax import lax
from jax.experimental import pallas as pl
from jax.experimental.pallas import tpu as pltpu
from jax.experimental.pallas import tpu_sc as plsc

N = 10000
E = 320000
D = 128
NC = 2
NS = 16
NW = NC * NS
KB = 128
NCH = 80
SEG = 16
CHUNKS = NW * NCH
NCH0 = 80
NCH1 = 80
E_PAD = CHUNKS * KB
N_PAD = N + 16
STEP = 624
STRIPE = 640
CW = 128

_mesh = plsc.VectorSubcoreMesh(
    core_axis_name="c", subcore_axis_name="s", num_cores=NC, num_subcores=NS
)


@functools.partial(
    pl.kernel,
    out_type=jax.ShapeDtypeStruct((NC * N, CW), jnp.float32),
    mesh=_mesh,
    scratch_types=[
        pltpu.VMEM((NCH, KB), jnp.int32),
        pltpu.VMEM((KB, CW), jnp.float32),
        pltpu.VMEM_SHARED((N_PAD, CW), jnp.float32),
    ],
)
def _sc_cnt(dst_hbm, z16_hbm, ones_hbm, cnt_out, dst_v, ones_v, cnt_s):
    cid = lax.axis_index("c")
    sid = lax.axis_index("s")
    wid = cid * NS + sid
    r0 = sid * STEP
    pltpu.sync_copy(z16_hbm.at[pl.ds(r0, STRIPE)], cnt_s.at[pl.ds(r0, STRIPE)])
    pltpu.sync_copy(ones_hbm, ones_v)
    pltpu.sync_copy(dst_hbm.at[pl.ds(wid * NCH, NCH)], dst_v)
    plsc.subcore_barrier()

    def step(j, carry):
        pltpu.sync_copy(ones_v, cnt_s.at[dst_v.at[j]], add=True)
        return carry

    lax.fori_loop(0, NCH, step, 0)
    plsc.subcore_barrier()
    o0 = cid * N + r0
    pltpu.sync_copy(cnt_s.at[pl.ds(r0, STRIPE)], cnt_out.at[pl.ds(o0, STRIPE)])


@functools.partial(
    pl.kernel,
    out_type=jax.ShapeDtypeStruct((NC * N, D), jnp.float32),
    mesh=_mesh,
    scratch_types=[
        pltpu.VMEM((SEG, KB), jnp.int32),
        pltpu.VMEM((SEG, KB), jnp.int32),
        pltpu.VMEM((KB, D), jnp.float32),
        pltpu.VMEM((KB, D), jnp.float32),
        pltpu.VMEM_SHARED((N_PAD, D), jnp.float32),
        pltpu.SemaphoreType.DMA,
        pltpu.SemaphoreType.DMA,
        pltpu.SemaphoreType.DMA,
    ],
)
def _sc_agg(x_hbm, src_hbm, dst_hbm, z128_hbm, agg_out, src_seg, dst_seg,
            rows0, rows1, agg_s, gs0, gs1, ss):
    cid = lax.axis_index("c")
    sid = lax.axis_index("s")
    r0 = sid * STEP
    pltpu.sync_copy(z128_hbm.at[pl.ds(r0, STRIPE)], agg_s.at[pl.ds(r0, STRIPE)])
    plsc.subcore_barrier()

    rows = (rows0, rows1)
    gsems = (gs0, gs1)

    def run_chunks(cbase, nseg):
        def seg_body(s, carry):
            base = cbase + s * SEG
            pltpu.sync_copy(src_hbm.at[pl.ds(base, SEG)], src_seg)
            pltpu.sync_copy(dst_hbm.at[pl.ds(base, SEG)], dst_seg)

            def step2(i, c2):
                j0 = 2 * i
                dgs = [
                    pltpu.async_copy(x_hbm.at[src_seg.at[j0 + b]], rows[b],
                                     gsems[b])
                    for b in range(2)
                ]
                dss = []
                for b in range(2):
                    dgs[b].wait()
                    dss.append(
                        pltpu.async_copy(rows[b],
                                         agg_s.at[dst_seg.at[j0 + b]],
                                         ss, add=True))
                for d in dss:
                    d.wait()
                return c2

            lax.fori_loop(0, SEG // 2, step2, 0)
            return carry

        lax.fori_loop(0, nseg, seg_body, 0)

    @pl.when(cid == 0)
    def _():
        run_chunks(sid * NCH0, NCH0 // SEG)

    if NCH1:
        @pl.when(cid == 1)
        def _():
            run_chunks(NS * NCH0 + sid * NCH1, NCH1 // SEG)

    plsc.subcore_barrier()
    o0 = cid * N + r0
    pltpu.sync_copy(agg_s.at[pl.ds(r0, STRIPE)], agg_out.at[pl.ds(o0, STRIPE)])


BR = 2000


def _dense_body(aggA, aggB, cntA, cntB, xin, wlT, b, wrT, o):
    c = cntA[:, 0:1] + cntB[:, 0:1]
    mean = (aggA[...] + aggB[...]) / jnp.maximum(c, 1.0)
    acc = jnp.dot(mean, wlT[...], preferred_element_type=jnp.float32,
                  precision=lax.Precision.HIGHEST)
    acc = acc + jnp.dot(xin[...], wrT[...], preferred_element_type=jnp.float32,
                        precision=lax.Precision.HIGHEST)
    o[...] = jnp.maximum(acc + b[...], 0.0)


def _dense(agg, cnt, xin, wlT, b2d, wrT):
    nb = N // BR
    return pl.pallas_call(
        _dense_body,
        grid=(nb,),
        in_specs=[
            pl.BlockSpec((BR, D), lambda i: (i, 0)),
            pl.BlockSpec((BR, D), lambda i: (i + nb, 0)),
            pl.BlockSpec((BR, CW), lambda i: (i, 0)),
            pl.BlockSpec((BR, CW), lambda i: (i + nb, 0)),
            pl.BlockSpec((BR, D), lambda i: (i, 0)),
            pl.BlockSpec((D, D), lambda i: (0, 0)),
            pl.BlockSpec((1, D), lambda i: (0, 0)),
            pl.BlockSpec((D, D), lambda i: (0, 0)),
        ],
        out_specs=pl.BlockSpec((BR, D), lambda i: (i, 0)),
        out_shape=jax.ShapeDtypeStruct((N, D), jnp.float32),
    )(agg, agg, cnt, cnt, xin, wlT, b2d, wrT)


def kernel(x, edge_index, W1l, b1, W1r, W2l, b2, W2r):
    pad = E_PAD - E
    pad_ix = jnp.arange(pad, dtype=jnp.int32)
    src = jnp.concatenate([edge_index[0], pad_ix % N])
    dst = jnp.concatenate([edge_index[1], N + pad_ix % (N_PAD - N)])
    src3 = src.reshape(CHUNKS, KB)
    dst3 = dst.reshape(CHUNKS, KB)
    z128 = jnp.zeros((N, D), jnp.float32)
    ones = jnp.ones((KB, CW), jnp.float32)

    cnt = _sc_cnt(dst3, z128, ones)
    agg1 = _sc_agg(x, src3, dst3, z128)
    h = _dense(agg1, cnt, x, W1l.T, b1.reshape(1, D), W1r.T)
    agg2 = _sc_agg(h, src3, dst3, z128)
    return _dense(agg2, cnt, h, W2l.T, b2.reshape(1, D), W2r.T)


# speedup vs baseline: 3.6301x; 1.2154x over previous
import functools

import jax
import jax.numpy as jnp
from jax import lax
from jax.experimental import pallas as pl
from jax.experimental.pallas import tpu as pltpu
from jax.experimental.pallas import tpu_sc as plsc

N = 10000
E = 320000
D = 128
NC = 2
NS = 16
NW = NC * NS
KB = 80
NCH = 128
SEG = 32
NSEG = NCH // SEG
CHUNKS = NW * NCH
E_PAD = CHUNKS * KB
N_PAD = N + 16
STEP = 624
STRIPE = 640
NSLOT = 4

_mesh = plsc.VectorSubcoreMesh(
    core_axis_name="c", subcore_axis_name="s", num_cores=NC, num_subcores=NS
)


@functools.partial(
    pl.kernel,
    out_type=jax.ShapeDtypeStruct((NC * N, D), jnp.float32),
    mesh=_mesh,
    scratch_types=[
        pltpu.VMEM((NCH, KB), jnp.int32),
        pltpu.VMEM((KB, D), jnp.float32),
        pltpu.VMEM_SHARED((N_PAD, D), jnp.float32),
    ],
)
def _sc_cnt(dst_hbm, z128_hbm, ones_hbm, cnt_out, dst_v, ones_v, cnt_s):
    cid = lax.axis_index("c")
    sid = lax.axis_index("s")
    wid = cid * NS + sid
    r0 = sid * STEP
    pltpu.sync_copy(z128_hbm.at[pl.ds(r0, STRIPE)],
                    cnt_s.at[pl.ds(r0, STRIPE)])
    pltpu.sync_copy(ones_hbm, ones_v)
    pltpu.sync_copy(dst_hbm.at[pl.ds(wid * NCH, NCH)], dst_v)
    plsc.subcore_barrier()

    def step(j, carry):
        pltpu.sync_copy(ones_v, cnt_s.at[dst_v.at[j]], add=True)
        return carry

    lax.fori_loop(0, NCH, step, 0)
    plsc.subcore_barrier()
    o0 = cid * N + r0
    pltpu.sync_copy(cnt_s.at[pl.ds(r0, STRIPE)], cnt_out.at[pl.ds(o0, STRIPE)])


@functools.partial(
    pl.kernel,
    out_type=jax.ShapeDtypeStruct((NC * N, D), jnp.float32),
    mesh=_mesh,
    scratch_types=[
        pltpu.VMEM((SEG, KB), jnp.int32),
        pltpu.VMEM((SEG, KB), jnp.int32),
        [pltpu.VMEM((KB, D), jnp.float32)] * NSLOT,
        [pltpu.SemaphoreType.DMA] * NSLOT,
        [pltpu.SemaphoreType.DMA] * NSLOT,
        pltpu.VMEM_SHARED((N_PAD, D), jnp.float32),
    ],
)
def _sc_agg(x_hbm, src_hbm, dst_hbm, z128_hbm, agg_out, src_seg, dst_seg,
            rows, gsems, ssems, agg_s):
    cid = lax.axis_index("c")
    sid = lax.axis_index("s")
    wid = cid * NS + sid
    r0 = sid * STEP
    pltpu.sync_copy(z128_hbm.at[pl.ds(r0, STRIPE)],
                    agg_s.at[pl.ds(r0, STRIPE)])
    plsc.subcore_barrier()

    def gather(b):
        return pltpu.async_copy(x_hbm.at[src_seg.at[b]], rows[b % NSLOT],
                                gsems[b % NSLOT])

    def scatter(b):
        return pltpu.async_copy(rows[b % NSLOT], agg_s.at[dst_seg.at[b]],
                                ssems[b % NSLOT], add=True)

    def seg_body(s, carry):
        base = wid * NCH + s * SEG
        pltpu.sync_copy(src_hbm.at[pl.ds(base, SEG)], src_seg)
        pltpu.sync_copy(dst_hbm.at[pl.ds(base, SEG)], dst_seg)
        dg = {0: gather(0), 1: gather(1)}
        ds = {}
        for b in range(SEG):
            if b >= 2:
                ds[b - 2].wait()
            if b + 2 < SEG:
                dg[b + 2] = gather(b + 2)
            dg[b].wait()
            ds[b] = scatter(b)
        ds[SEG - 2].wait()
        ds[SEG - 1].wait()
        return carry

    lax.fori_loop(0, NSEG, seg_body, 0)
    plsc.subcore_barrier()
    o0 = cid * N + r0
    pltpu.sync_copy(agg_s.at[pl.ds(r0, STRIPE)], agg_out.at[pl.ds(o0, STRIPE)])


BR = 2000


def _dense_body(aggA, aggB, cntA, cntB, xin, wlT, b, wrT, o):
    c = cntA[:, 0:1] + cntB[:, 0:1]
    mean = (aggA[...] + aggB[...]) / jnp.maximum(c, 1.0)
    acc = jnp.dot(mean, wlT[...], preferred_element_type=jnp.float32,
                  precision=lax.Precision.HIGHEST)
    acc = acc + jnp.dot(xin[...], wrT[...], preferred_element_type=jnp.float32,
                        precision=lax.Precision.HIGHEST)
    o[...] = jnp.maximum(acc + b[...], 0.0)


def _dense(agg, cnt, xin, wlT, b2d, wrT):
    nb = N // BR
    return pl.pallas_call(
        _dense_body,
        grid=(nb,),
        in_specs=[
            pl.BlockSpec((BR, D), lambda i: (i, 0)),
            pl.BlockSpec((BR, D), lambda i: (i + nb, 0)),
            pl.BlockSpec((BR, D), lambda i: (i, 0)),
            pl.BlockSpec((BR, D), lambda i: (i + nb, 0)),
            pl.BlockSpec((BR, D), lambda i: (i, 0)),
            pl.BlockSpec((D, D), lambda i: (0, 0)),
            pl.BlockSpec((1, D), lambda i: (0, 0)),
            pl.BlockSpec((D, D), lambda i: (0, 0)),
        ],
        out_specs=pl.BlockSpec((BR, D), lambda i: (i, 0)),
        out_shape=jax.ShapeDtypeStruct((N, D), jnp.float32),
    )(agg, agg, cnt, cnt, xin, wlT, b2d, wrT)


def kernel(x, edge_index, W1l, b1, W1r, W2l, b2, W2r):
    pad = E_PAD - E
    pad_ix = jnp.arange(pad, dtype=jnp.int32)
    src = jnp.concatenate([edge_index[0], pad_ix % N])
    dst = jnp.concatenate([edge_index[1], N + pad_ix % (N_PAD - N)])
    src2 = src.reshape(CHUNKS, KB)
    dst2 = dst.reshape(CHUNKS, KB)
    z128 = jnp.zeros((N, D), jnp.float32)
    ones = jnp.ones((KB, D), jnp.float32)

    cnt = _sc_cnt(dst2, z128, ones)
    agg1 = _sc_agg(x, src2, dst2, z128)
    h = _dense(agg1, cnt, x, W1l.T, b1.reshape(1, D), W1r.T)
    agg2 = _sc_agg(h, src2, dst2, z128)
    return _dense(agg2, cnt, h, W2l.T, b2.reshape(1, D), W2r.T)


# speedup vs baseline: 3.6406x; 1.0029x over previous
import functools

import jax
import jax.numpy as jnp
from jax import lax
from jax.experimental import pallas as pl
from jax.experimental.pallas import tpu as pltpu
from jax.experimental.pallas import tpu_sc as plsc

N = 10000
E = 320000
D = 128
NC = 2
NS = 16
NW = NC * NS
KB = 80
NCH = 128
SEG = 32
NSEG = NCH // SEG
CHUNKS = NW * NCH
E_PAD = CHUNKS * KB
N_PAD = N + 16
STEP = 624
STRIPE = 640
NSLOT = 4

_mesh = plsc.VectorSubcoreMesh(
    core_axis_name="c", subcore_axis_name="s", num_cores=NC, num_subcores=NS
)


@functools.partial(
    pl.kernel,
    out_type=jax.ShapeDtypeStruct((NC * N, D), jnp.float32),
    mesh=_mesh,
    scratch_types=[
        pltpu.VMEM((NCH, KB), jnp.int32),
        pltpu.VMEM((KB, D), jnp.float32),
        [pltpu.SemaphoreType.DMA] * NSLOT,
        pltpu.VMEM_SHARED((N_PAD, D), jnp.float32),
    ],
)
def _sc_cnt(dst_hbm, z128_hbm, ones_hbm, cnt_out, dst_v, ones_v, sems, cnt_s):
    cid = lax.axis_index("c")
    sid = lax.axis_index("s")
    wid = cid * NS + sid
    r0 = sid * STEP
    pltpu.sync_copy(z128_hbm.at[pl.ds(r0, STRIPE)],
                    cnt_s.at[pl.ds(r0, STRIPE)])
    pltpu.sync_copy(ones_hbm, ones_v)
    pltpu.sync_copy(dst_hbm.at[pl.ds(wid * NCH, NCH)], dst_v)
    plsc.subcore_barrier()

    def step(i, carry):
        dss = [
            pltpu.async_copy(ones_v, cnt_s.at[dst_v.at[NSLOT * i + b]],
                             sems[b], add=True)
            for b in range(NSLOT)
        ]
        for d in dss:
            d.wait()
        return carry

    lax.fori_loop(0, NCH // NSLOT, step, 0)
    plsc.subcore_barrier()
    o0 = cid * N + r0
    pltpu.sync_copy(cnt_s.at[pl.ds(r0, STRIPE)], cnt_out.at[pl.ds(o0, STRIPE)])


@functools.partial(
    pl.kernel,
    out_type=jax.ShapeDtypeStruct((NC * N, D), jnp.float32),
    mesh=_mesh,
    scratch_types=[
        pltpu.VMEM((SEG, KB), jnp.int32),
        pltpu.VMEM((SEG, KB), jnp.int32),
        [pltpu.VMEM((KB, D), jnp.float32)] * NSLOT,
        [pltpu.SemaphoreType.DMA] * NSLOT,
        [pltpu.SemaphoreType.DMA] * NSLOT,
        pltpu.VMEM_SHARED((N_PAD, D), jnp.float32),
    ],
)
def _sc_agg(x_hbm, src_hbm, dst_hbm, z128_hbm, agg_out, src_seg, dst_seg,
            rows, gsems, ssems, agg_s):
    cid = lax.axis_index("c")
    sid = lax.axis_index("s")
    wid = cid * NS + sid
    r0 = sid * STEP
    pltpu.sync_copy(z128_hbm.at[pl.ds(r0, STRIPE)],
                    agg_s.at[pl.ds(r0, STRIPE)])
    plsc.subcore_barrier()

    def gather(b):
        return pltpu.async_copy(x_hbm.at[src_seg.at[b]], rows[b % NSLOT],
                                gsems[b % NSLOT])

    def scatter(b):
        return pltpu.async_copy(rows[b % NSLOT], agg_s.at[dst_seg.at[b]],
                                ssems[b % NSLOT], add=True)

    def seg_body(s, carry):
        base = wid * NCH + s * SEG
        pltpu.sync_copy(src_hbm.at[pl.ds(base, SEG)], src_seg)
        pltpu.sync_copy(dst_hbm.at[pl.ds(base, SEG)], dst_seg)
        dg = {0: gather(0), 1: gather(1)}
        ds = {}
        for b in range(SEG):
            if b >= 2:
                ds[b - 2].wait()
            if b + 2 < SEG:
                dg[b + 2] = gather(b + 2)
            dg[b].wait()
            ds[b] = scatter(b)
        ds[SEG - 2].wait()
        ds[SEG - 1].wait()
        return carry

    lax.fori_loop(0, NSEG, seg_body, 0)
    plsc.subcore_barrier()
    o0 = cid * N + r0
    pltpu.sync_copy(agg_s.at[pl.ds(r0, STRIPE)], agg_out.at[pl.ds(o0, STRIPE)])


BR = 2000


def _dense_body(aggA, aggB, cntA, cntB, xin, wlT, b, wrT, o):
    c = cntA[:, 0:1] + cntB[:, 0:1]
    mean = (aggA[...] + aggB[...]) / jnp.maximum(c, 1.0)
    acc = jnp.dot(mean, wlT[...], preferred_element_type=jnp.float32,
                  precision=lax.Precision.HIGHEST)
    acc = acc + jnp.dot(xin[...], wrT[...], preferred_element_type=jnp.float32,
                        precision=lax.Precision.HIGHEST)
    o[...] = jnp.maximum(acc + b[...], 0.0)


def _dense(agg, cnt, xin, wlT, b2d, wrT):
    nb = N // BR
    return pl.pallas_call(
        _dense_body,
        grid=(nb,),
        in_specs=[
            pl.BlockSpec((BR, D), lambda i: (i, 0)),
            pl.BlockSpec((BR, D), lambda i: (i + nb, 0)),
            pl.BlockSpec((BR, D), lambda i: (i, 0)),
            pl.BlockSpec((BR, D), lambda i: (i + nb, 0)),
            pl.BlockSpec((BR, D), lambda i: (i, 0)),
            pl.BlockSpec((D, D), lambda i: (0, 0)),
            pl.BlockSpec((1, D), lambda i: (0, 0)),
            pl.BlockSpec((D, D), lambda i: (0, 0)),
        ],
        out_specs=pl.BlockSpec((BR, D), lambda i: (i, 0)),
        out_shape=jax.ShapeDtypeStruct((N, D), jnp.float32),
    )(agg, agg, cnt, cnt, xin, wlT, b2d, wrT)


def kernel(x, edge_index, W1l, b1, W1r, W2l, b2, W2r):
    pad = E_PAD - E
    pad_ix = jnp.arange(pad, dtype=jnp.int32)
    src = jnp.concatenate([edge_index[0], pad_ix % N])
    dst = jnp.concatenate([edge_index[1], N + pad_ix % (N_PAD - N)])
    src2 = src.reshape(CHUNKS, KB)
    dst2 = dst.reshape(CHUNKS, KB)
    z128 = jnp.zeros((N, D), jnp.float32)
    ones = jnp.ones((KB, D), jnp.float32)

    cnt = _sc_cnt(dst2, z128, ones)
    agg1 = _sc_agg(x, src2, dst2, z128)
    h = _dense(agg1, cnt, x, W1l.T, b1.reshape(1, D), W1r.T)
    agg2 = _sc_agg(h, src2, dst2, z128)
    return _dense(agg2, cnt, h, W2l.T, b2.reshape(1, D), W2r.T)
